# baseline (speedup 1.0000x reference)
import functools

import jax
import jax.numpy as jnp
from jax import lax
from jax.experimental import pallas as pl
from jax.experimental.pallas import tpu as pltpu
from jax.experimental.pallas import tpu_sc as plsc

_B, _P, _O, _D = 16384, 97, 13, 128
_NW = 32
_BPW = _B // _NW
_ROWS = _BPW * 4
_CH = 128
_NCH = _ROWS // _CH
_L = 16

_mesh = plsc.VectorSubcoreMesh(core_axis_name="c", subcore_axis_name="s")


@functools.partial(
    pl.kernel,
    mesh=_mesh,
    out_type=jax.ShapeDtypeStruct((_B * 4, _D), jnp.float32),
    scratch_types=[
        pltpu.VMEM((_BPW,), jnp.int32),
        pltpu.VMEM((_BPW,), jnp.int32),
        pltpu.VMEM((_BPW,), jnp.int32),
        pltpu.VMEM((_ROWS,), jnp.int32),
        pltpu.VMEM_SHARED((_P + _O, _D), jnp.float32),
        pltpu.VMEM((2 * _CH, _D), jnp.float32),
        pltpu.VMEM((2 * _CH, _D), jnp.float32),
        pltpu.SemaphoreType.DMA,
        pltpu.SemaphoreType.DMA,
        pltpu.SemaphoreType.DMA,
        pltpu.SemaphoreType.DMA,
    ],
)
def _former(x_hbm, op_hbm, y_hbm, table_hbm, out_hbm,
            xv, ov, yv, cidx, table_v, buf0, buf1,
            gs0, gs1, os0, os1):
    wid = lax.axis_index("s") * 2 + lax.axis_index("c")
    base = wid * _BPW
    pltpu.sync_copy(x_hbm.at[pl.ds(base, _BPW)], xv)
    pltpu.sync_copy(op_hbm.at[pl.ds(base, _BPW)], ov)
    pltpu.sync_copy(y_hbm.at[pl.ds(base, _BPW)], yv)
    @pl.when(lax.axis_index("s") == 0)
    def _stage_table():
        pltpu.sync_copy(table_hbm, table_v)
    plsc.subcore_barrier()

    lanes = lax.iota(jnp.int32, _L)
    rep = lanes >> 2
    slot = lanes & 3
    for j in range(_BPW // _L):
        xb = xv[pl.ds(j * _L, _L)]
        ob = ov[pl.ds(j * _L, _L)] + _P
        yb = yv[pl.ds(j * _L, _L)]
        for k in range(4):
            b = rep + k * 4
            xs = xb.at[b].get(mode="promise_in_bounds")
            os_ = ob.at[b].get(mode="promise_in_bounds")
            ys = yb.at[b].get(mode="promise_in_bounds")
            v = jnp.where(slot == 0, xs,
                jnp.where(slot == 1, os_,
                jnp.where(slot == 2, ys, _P)))
            cidx[pl.ds((j * 4 + k) * _L, _L)] = v

    bufs = (buf0, buf1)
    gsems = (gs0, gs1)
    osems = (os0, os1)
    obase = base * 4
    nsc = _NCH // 2
    gd = [None] * _NCH
    od = [None] * nsc

    def _gathers(J):
        b = bufs[J % 2]
        s = gsems[J % 2]
        for h in range(2):
            j = 2 * J + h
            gd[j] = pltpu.async_copy(
                table_v.at[cidx.at[pl.ds(j * _CH, _CH)]],
                b.at[pl.ds(h * _CH, _CH)], s)

    _gathers(0)
    for J in range(nsc):
        if J + 1 < nsc:
            if J >= 1:
                od[J - 1].wait()
            _gathers(J + 1)
        gd[2 * J].wait()
        gd[2 * J + 1].wait()
        od[J] = pltpu.async_copy(
            bufs[J % 2], out_hbm.at[pl.ds(obase + J * 2 * _CH, 2 * _CH)],
            osems[J % 2])
    od[nsc - 2].wait()
    od[nsc - 1].wait()


def kernel(x_idx, op_idx, y_idx, number_emb, op_emb):
    table = jnp.concatenate([number_emb, op_emb], axis=0)
    out = _former(x_idx.astype(jnp.int32), op_idx.astype(jnp.int32),
                  y_idx.astype(jnp.int32), table)
    return out.reshape(_B, 4, _D)


# speedup vs baseline: 1.0392x; 1.0392x over previous
import functools

import jax
import jax.numpy as jnp
from jax import lax
from jax.experimental import pallas as pl
from jax.experimental.pallas import tpu as pltpu
from jax.experimental.pallas import tpu_sc as plsc

_B, _P, _O, _D = 16384, 97, 13, 128
_NW = 32
_BPW = _B // _NW
_ROWS = _BPW * 4
_CH = 128
_NCH = _ROWS // _CH
_L = 16

_mesh = plsc.VectorSubcoreMesh(core_axis_name="c", subcore_axis_name="s")


@functools.partial(
    pl.kernel,
    mesh=_mesh,
    out_type=jax.ShapeDtypeStruct((_B * 4, _D), jnp.float32),
    scratch_types=[
        pltpu.VMEM((_BPW,), jnp.int32),
        pltpu.VMEM((_BPW,), jnp.int32),
        pltpu.VMEM((_BPW,), jnp.int32),
        pltpu.VMEM((_ROWS,), jnp.int32),
        pltpu.VMEM_SHARED((_P + _O, _D), jnp.float32),
        pltpu.VMEM((_CH, _D), jnp.float32),
        pltpu.VMEM((_CH, _D), jnp.float32),
        pltpu.VMEM((_CH, _D), jnp.float32),
        pltpu.VMEM((_CH, _D), jnp.float32),
        pltpu.SemaphoreType.DMA,
        pltpu.SemaphoreType.DMA,
        pltpu.SemaphoreType.DMA,
        pltpu.SemaphoreType.DMA,
        pltpu.SemaphoreType.DMA,
        pltpu.SemaphoreType.DMA,
        pltpu.SemaphoreType.DMA,
        pltpu.SemaphoreType.DMA,
    ],
)
def _former(x_hbm, op_hbm, y_hbm, table_hbm, out_hbm,
            xv, ov, yv, cidx, table_v, buf0, buf1, buf2, buf3,
            gs0, gs1, gs2, gs3, os0, os1, os2, os3):
    wid = lax.axis_index("s") * 2 + lax.axis_index("c")
    base = wid * _BPW
    pltpu.sync_copy(x_hbm.at[pl.ds(base, _BPW)], xv)
    pltpu.sync_copy(op_hbm.at[pl.ds(base, _BPW)], ov)
    pltpu.sync_copy(y_hbm.at[pl.ds(base, _BPW)], yv)
    @pl.when(lax.axis_index("s") == 0)
    def _stage_table():
        pltpu.sync_copy(table_hbm, table_v)
    plsc.subcore_barrier()

    lanes = lax.iota(jnp.int32, _L)
    rep = lanes >> 2
    slot = lanes & 3
    for j in range(_BPW // _L):
        xb = xv[pl.ds(j * _L, _L)]
        ob = ov[pl.ds(j * _L, _L)] + _P
        yb = yv[pl.ds(j * _L, _L)]
        for k in range(4):
            b = rep + k * 4
            xs = xb.at[b].get(mode="promise_in_bounds")
            os_ = ob.at[b].get(mode="promise_in_bounds")
            ys = yb.at[b].get(mode="promise_in_bounds")
            v = jnp.where(slot == 0, xs,
                jnp.where(slot == 1, os_,
                jnp.where(slot == 2, ys, _P)))
            cidx[pl.ds((j * 4 + k) * _L, _L)] = v

    bufs = (buf0, buf1, buf2, buf3)
    gsems = (gs0, gs1, gs2, gs3)
    osems = (os0, os1, os2, os3)
    nbuf = 4
    obase = base * 4
    gd = [None] * _NCH
    od = [None] * _NCH

    def _gather(j):
        gd[j] = pltpu.async_copy(
            table_v.at[cidx.at[pl.ds(j * _CH, _CH)]],
            bufs[j % nbuf], gsems[j % nbuf])

    for j in range(nbuf - 1):
        _gather(j)
    for j in range(_NCH):
        ahead = j + nbuf - 1
        if ahead < _NCH:
            if j >= 1:
                od[j - 1].wait()
            _gather(ahead)
        gd[j].wait()
        od[j] = pltpu.async_copy(
            bufs[j % nbuf], out_hbm.at[pl.ds(obase + j * _CH, _CH)],
            osems[j % nbuf])
    for j in range(_NCH - nbuf, _NCH):
        if j >= 0 and od[j] is not None:
            od[j].wait()


def kernel(x_idx, op_idx, y_idx, number_emb, op_emb):
    table = jnp.concatenate([number_emb, op_emb], axis=0)
    out = _former(x_idx.astype(jnp.int32), op_idx.astype(jnp.int32),
                  y_idx.astype(jnp.int32), table)
    return out.reshape(_B, 4, _D)


# speedup vs baseline: 1.0713x; 1.0310x over previous
import functools

import jax
import jax.numpy as jnp
from jax import lax
from jax.experimental import pallas as pl
from jax.experimental.pallas import tpu as pltpu
from jax.experimental.pallas import tpu_sc as plsc

_B, _P, _O, _D = 16384, 97, 13, 128
_NW = 32
_BPW = _B // _NW
_CH = 128
_NST = _BPW // _CH
_L = 16

_mesh = plsc.VectorSubcoreMesh(core_axis_name="c", subcore_axis_name="s")


@functools.partial(
    pl.kernel,
    mesh=_mesh,
    out_type=jax.ShapeDtypeStruct((_B, 4, _D), jnp.float32),
    scratch_types=[
        pltpu.VMEM((_BPW,), jnp.int32),
        pltpu.VMEM((_BPW,), jnp.int32),
        pltpu.VMEM((_BPW,), jnp.int32),
        pltpu.VMEM((_CH,), jnp.int32),
        pltpu.VMEM_SHARED((_P + _O, _D), jnp.float32),
        pltpu.VMEM((_CH, _D), jnp.float32),
        pltpu.VMEM((_CH, _D), jnp.float32),
        pltpu.VMEM((_CH, _D), jnp.float32),
        pltpu.VMEM((_CH, _D), jnp.float32),
        pltpu.VMEM((_CH, _D), jnp.float32),
        pltpu.VMEM((_CH, _D), jnp.float32),
        pltpu.VMEM((_CH, _D), jnp.float32),
        pltpu.SemaphoreType.DMA,
        pltpu.SemaphoreType.DMA,
        pltpu.SemaphoreType.DMA,
        pltpu.SemaphoreType.DMA,
        pltpu.SemaphoreType.DMA,
    ],
)
def _former(x_hbm, op_hbm, y_hbm, table_hbm, out_hbm,
            xv, ov, yv, eqi, table_v,
            bx0, bx1, bo0, bo1, by0, by1, beq,
            gs0, gs1, ws0, ws1, wse):
    wid = lax.axis_index("s") * 2 + lax.axis_index("c")
    base = wid * _BPW
    pltpu.sync_copy(x_hbm.at[pl.ds(base, _BPW)], xv)
    pltpu.sync_copy(op_hbm.at[pl.ds(base, _BPW)], ov)
    pltpu.sync_copy(y_hbm.at[pl.ds(base, _BPW)], yv)
    @pl.when(lax.axis_index("s") == 0)
    def _stage_table():
        pltpu.sync_copy(table_hbm, table_v)
    for j in range(_BPW // _L):
        ov[pl.ds(j * _L, _L)] = ov[pl.ds(j * _L, _L)] + _P
    for j in range(_CH // _L):
        eqi[pl.ds(j * _L, _L)] = jnp.full((_L,), _P, jnp.int32)
    plsc.subcore_barrier()

    pltpu.async_copy(table_v.at[eqi], beq, gs0).wait()

    bxs, bos, bys = (bx0, bx1), (bo0, bo1), (by0, by1)
    gsems = (gs0, gs1)
    wsems = (ws0, ws1)
    gd = [None] * _NST
    wd = [None] * _NST

    def _gathers(t):
        p = t % 2
        sl = pl.ds(t * _CH, _CH)
        gd[t] = (
            pltpu.async_copy(table_v.at[xv.at[sl]], bxs[p], gsems[p]),
            pltpu.async_copy(table_v.at[ov.at[sl]], bos[p], gsems[p]),
            pltpu.async_copy(table_v.at[yv.at[sl]], bys[p], gsems[p]),
        )

    def _writes(t):
        p = t % 2
        rows = pl.ds(base + t * _CH, _CH)
        wd[t] = (
            pltpu.async_copy(bxs[p], out_hbm.at[rows, 0], wsems[p]),
            pltpu.async_copy(bos[p], out_hbm.at[rows, 1], wsems[p]),
            pltpu.async_copy(bys[p], out_hbm.at[rows, 2], wsems[p]),
            pltpu.async_copy(beq, out_hbm.at[rows, 3], wse),
        )

    _gathers(0)
    for t in range(_NST):
        if t + 1 < _NST:
            if t >= 1:
                for d in wd[t - 1][:3]:
                    d.wait()
            _gathers(t + 1)
        for d in gd[t]:
            d.wait()
        _writes(t)
    for t in (_NST - 2, _NST - 1):
        for d in wd[t][:3]:
            d.wait()
    for t in range(_NST):
        wd[t][3].wait()


def kernel(x_idx, op_idx, y_idx, number_emb, op_emb):
    table = jnp.concatenate([number_emb, op_emb], axis=0)
    return _former(x_idx.astype(jnp.int32), op_idx.astype(jnp.int32),
                   y_idx.astype(jnp.int32), table)
